# baseline (speedup 1.0000x reference)
import functools

import jax
import jax.numpy as jnp
from jax import lax
from jax.experimental import pallas as pl
from jax.experimental.pallas import tpu as pltpu
from jax.experimental.pallas import tpu_sc as plsc

B0, B1, B2 = 16, 128, 30
E_PAD = 32
NP = B0 * B1 * E_PAD
D = 512
NC, NS = 2, 16
NW = NC * NS

CHUNK = 64
BLK = 4096
SIZES = (8192, 16384, 20480, 20480)
BASES = tuple(sum(SIZES[:s]) for s in range(len(SIZES)))


def _sc_gather_slice(ids, table, posmap, s):
    mesh = plsc.VectorSubcoreMesh(core_axis_name="c", subcore_axis_name="s")
    slice_base = BASES[s]
    n_s = SIZES[s]
    b_per_w = n_s // NW
    n_chunks = b_per_w // CHUNK

    @functools.partial(
        pl.kernel,
        mesh=mesh,
        out_type=(
            jax.ShapeDtypeStruct((n_s, D), jnp.float32),
            jax.ShapeDtypeStruct((n_s,), jnp.int32),
        ),
        scratch_types=[
            pltpu.VMEM((b_per_w,), jnp.int32),
            pltpu.VMEM((2, CHUNK, D), jnp.float32),
            pltpu.VMEM((2, CHUNK), jnp.int32),
            pltpu.SemaphoreType.DMA,
            pltpu.SemaphoreType.DMA,
            pltpu.SemaphoreType.DMA,
            pltpu.SemaphoreType.DMA,
        ],
    )
    def k(ids_hbm, table_hbm, posmap_hbm, rows_out, pos_out,
          idx_v, rows_v, pos_v, sem_r0, sem_r1, sem_p0, sem_p1):
        wid = lax.axis_index("s") * NC + lax.axis_index("c")
        base = wid * b_per_w
        sem_r = (sem_r0, sem_r1)
        sem_p = (sem_p0, sem_p1)
        pltpu.sync_copy(ids_hbm.at[pl.ds(slice_base + base, b_per_w)], idx_v)

        def start(ch):
            buf = ch % 2
            cp_r = pltpu.async_copy(
                table_hbm.at[idx_v.at[pl.ds(ch * CHUNK, CHUNK)]],
                rows_v.at[buf], sem_r[buf])
            cp_p = pltpu.async_copy(
                posmap_hbm.at[idx_v.at[pl.ds(ch * CHUNK, CHUNK)]],
                pos_v.at[buf], sem_p[buf])
            return cp_r, cp_p

        pend = start(0)
        for ch in range(n_chunks):
            buf = ch % 2
            nxt = pend
            if ch + 1 < n_chunks:
                pend_next = start(ch + 1)
            nxt[0].wait()
            nxt[1].wait()
            off = base + ch * CHUNK
            pltpu.sync_copy(rows_v.at[buf], rows_out.at[pl.ds(off, CHUNK)])
            pltpu.sync_copy(pos_v.at[buf], pos_out.at[pl.ds(off, CHUNK)])
            if ch + 1 < n_chunks:
                pend = pend_next

    return k(ids, table, posmap)


def _tc_body(prev_ref, g_ref, pos_ref, we_ref, pos16_ref, wp_ref, b_ref,
             out_ref):
    del prev_ref
    p16 = lax.dot_general(
        pos16_ref[...], wp_ref[...], (((1,), (1,)), ((), ())),
        preferred_element_type=jnp.float32) + b_ref[...]
    pos = pos_ref[0, 0, :]
    onehot = (pos[:, None] == lax.broadcasted_iota(
        jnp.int32, (BLK, 16), 1)).astype(jnp.float32)
    out_ref[...] = (
        lax.dot_general(g_ref[...], we_ref[...], (((1,), (1,)), ((), ())),
                        preferred_element_type=jnp.float32)
        + jnp.dot(onehot, p16, preferred_element_type=jnp.float32))


def _tc_project_slice(prev, rows_s, pos3_s, we, pos16, wp, b2, s):
    nb_s = SIZES[s] // BLK
    blk_base = BASES[s] // BLK
    in_specs = [
        pl.BlockSpec((BLK, D), lambda i: (i, 0)),
        pl.BlockSpec((1, 1, BLK), lambda i: (i, 0, 0)),
        pl.BlockSpec((D, D), lambda i: (0, 0)),
        pl.BlockSpec((16, 32), lambda i: (0, 0)),
        pl.BlockSpec((D, 32), lambda i: (0, 0)),
        pl.BlockSpec((1, D), lambda i: (0, 0)),
    ]
    if prev is None:
        return pl.pallas_call(
            lambda *refs: _tc_body(None, *refs),
            grid=(nb_s,),
            in_specs=in_specs,
            out_specs=pl.BlockSpec((BLK, D), lambda i: (i, 0)),
            out_shape=jax.ShapeDtypeStruct((NP, D), jnp.float32),
        )(rows_s, pos3_s, we, pos16, wp, b2)
    return pl.pallas_call(
        _tc_body,
        grid=(nb_s,),
        in_specs=[pl.BlockSpec(memory_space=pl.ANY)] + in_specs,
        out_specs=pl.BlockSpec(
            (BLK, D), lambda i, blk_base=blk_base: (blk_base + i, 0)),
        out_shape=jax.ShapeDtypeStruct((NP, D), jnp.float32),
        input_output_aliases={0: 0},
    )(prev, rows_s, pos3_s, we, pos16, wp, b2)


def kernel(entity_ids, entity_table, pos_table, entity_id_to_pos_index, W, b):
    filler = jnp.arange(NP, dtype=jnp.int32).reshape(B0, B1, E_PAD)
    padded = jnp.pad(entity_ids.astype(jnp.int32),
                     ((0, 0), (0, 0), (0, E_PAD - B2)))
    emask = (jnp.arange(E_PAD) < B2)[None, None, :]
    ids = jnp.where(emask, padded, filler).reshape(-1)
    posmap = entity_id_to_pos_index.astype(jnp.int32)

    we = W[:, :D]
    wp = jnp.pad(W[:, D:], ((0, 0), (0, 7)))
    pos16 = jnp.pad(pos_table[:16], ((0, 0), (0, 7)))
    b2 = b.reshape(1, D)

    gathered = [_sc_gather_slice(ids, entity_table, posmap, s)
                for s in range(len(SIZES))]

    out = None
    for s, (rows_s, pos_s) in enumerate(gathered):
        pos3_s = pos_s.reshape(SIZES[s] // BLK, 1, BLK)
        out = _tc_project_slice(out, rows_s, pos3_s, we, pos16, wp, b2, s)

    return out.reshape(B0, B1, E_PAD, D)[:, :, :B2, :]


# speedup vs baseline: 1.0022x; 1.0022x over previous
import functools

import jax
import jax.numpy as jnp
from jax import lax
from jax.experimental import pallas as pl
from jax.experimental.pallas import tpu as pltpu
from jax.experimental.pallas import tpu_sc as plsc

B0, B1, B2 = 16, 128, 30
E_PAD = 32
NP = B0 * B1 * E_PAD
D = 512
NC, NS = 2, 16
NW = NC * NS

CHUNK = 64
BLK = 4096
SIZES = (16384, 16384, 16384, 16384)
BASES = tuple(sum(SIZES[:s]) for s in range(len(SIZES)))


def _sc_gather_slice(ids, table, posmap, s):
    mesh = plsc.VectorSubcoreMesh(core_axis_name="c", subcore_axis_name="s")
    slice_base = BASES[s]
    n_s = SIZES[s]
    b_per_w = n_s // NW
    n_chunks = b_per_w // CHUNK

    @functools.partial(
        pl.kernel,
        mesh=mesh,
        out_type=(
            jax.ShapeDtypeStruct((n_s, D), jnp.float32),
            jax.ShapeDtypeStruct((n_s,), jnp.int32),
        ),
        scratch_types=[
            pltpu.VMEM((b_per_w,), jnp.int32),
            pltpu.VMEM((2, CHUNK, D), jnp.float32),
            pltpu.VMEM((2, CHUNK), jnp.int32),
            pltpu.SemaphoreType.DMA,
            pltpu.SemaphoreType.DMA,
            pltpu.SemaphoreType.DMA,
            pltpu.SemaphoreType.DMA,
        ],
    )
    def k(ids_hbm, table_hbm, posmap_hbm, rows_out, pos_out,
          idx_v, rows_v, pos_v, sem_r0, sem_r1, sem_p0, sem_p1):
        wid = lax.axis_index("s") * NC + lax.axis_index("c")
        base = wid * b_per_w
        sem_r = (sem_r0, sem_r1)
        sem_p = (sem_p0, sem_p1)
        pltpu.sync_copy(ids_hbm.at[pl.ds(slice_base + base, b_per_w)], idx_v)

        def start(ch):
            buf = ch % 2
            cp_r = pltpu.async_copy(
                table_hbm.at[idx_v.at[pl.ds(ch * CHUNK, CHUNK)]],
                rows_v.at[buf], sem_r[buf])
            cp_p = pltpu.async_copy(
                posmap_hbm.at[idx_v.at[pl.ds(ch * CHUNK, CHUNK)]],
                pos_v.at[buf], sem_p[buf])
            return cp_r, cp_p

        pend = start(0)
        for ch in range(n_chunks):
            buf = ch % 2
            nxt = pend
            if ch + 1 < n_chunks:
                pend_next = start(ch + 1)
            nxt[0].wait()
            nxt[1].wait()
            off = base + ch * CHUNK
            pltpu.sync_copy(rows_v.at[buf], rows_out.at[pl.ds(off, CHUNK)])
            pltpu.sync_copy(pos_v.at[buf], pos_out.at[pl.ds(off, CHUNK)])
            if ch + 1 < n_chunks:
                pend = pend_next

    return k(ids, table, posmap)


def _tc_body(prev_ref, g_ref, pos_ref, we_ref, pos16_ref, wp_ref, b_ref,
             out_ref):
    del prev_ref
    p16 = lax.dot_general(
        pos16_ref[...], wp_ref[...], (((1,), (1,)), ((), ())),
        preferred_element_type=jnp.float32) + b_ref[...]
    pos = pos_ref[0, 0, :]
    onehot = (pos[:, None] == lax.broadcasted_iota(
        jnp.int32, (BLK, 16), 1)).astype(jnp.float32)
    out_ref[...] = (
        lax.dot_general(g_ref[...], we_ref[...], (((1,), (1,)), ((), ())),
                        preferred_element_type=jnp.float32)
        + jnp.dot(onehot, p16, preferred_element_type=jnp.float32))


def _tc_project_slice(prev, rows_s, pos3_s, we, pos16, wp, b2, s):
    nb_s = SIZES[s] // BLK
    blk_base = BASES[s] // BLK
    in_specs = [
        pl.BlockSpec((BLK, D), lambda i: (i, 0)),
        pl.BlockSpec((1, 1, BLK), lambda i: (i, 0, 0)),
        pl.BlockSpec((D, D), lambda i: (0, 0)),
        pl.BlockSpec((16, 32), lambda i: (0, 0)),
        pl.BlockSpec((D, 32), lambda i: (0, 0)),
        pl.BlockSpec((1, D), lambda i: (0, 0)),
    ]
    if prev is None:
        return pl.pallas_call(
            lambda *refs: _tc_body(None, *refs),
            grid=(nb_s,),
            in_specs=in_specs,
            out_specs=pl.BlockSpec((BLK, D), lambda i: (i, 0)),
            out_shape=jax.ShapeDtypeStruct((NP, D), jnp.float32),
        )(rows_s, pos3_s, we, pos16, wp, b2)
    return pl.pallas_call(
        _tc_body,
        grid=(nb_s,),
        in_specs=[pl.BlockSpec(memory_space=pl.ANY)] + in_specs,
        out_specs=pl.BlockSpec(
            (BLK, D), lambda i, blk_base=blk_base: (blk_base + i, 0)),
        out_shape=jax.ShapeDtypeStruct((NP, D), jnp.float32),
        input_output_aliases={0: 0},
    )(prev, rows_s, pos3_s, we, pos16, wp, b2)


def kernel(entity_ids, entity_table, pos_table, entity_id_to_pos_index, W, b):
    filler = jnp.arange(NP, dtype=jnp.int32).reshape(B0, B1, E_PAD)
    padded = jnp.pad(entity_ids.astype(jnp.int32),
                     ((0, 0), (0, 0), (0, E_PAD - B2)))
    emask = (jnp.arange(E_PAD) < B2)[None, None, :]
    ids = jnp.where(emask, padded, filler).reshape(-1)
    posmap = entity_id_to_pos_index.astype(jnp.int32)

    we = W[:, :D]
    wp = jnp.pad(W[:, D:], ((0, 0), (0, 7)))
    pos16 = jnp.pad(pos_table[:16], ((0, 0), (0, 7)))
    b2 = b.reshape(1, D)

    gathered = [_sc_gather_slice(ids, entity_table, posmap, s)
                for s in range(len(SIZES))]

    out = None
    for s, (rows_s, pos_s) in enumerate(gathered):
        pos3_s = pos_s.reshape(SIZES[s] // BLK, 1, BLK)
        out = _tc_project_slice(out, rows_s, pos3_s, we, pos16, wp, b2, s)

    return out.reshape(B0, B1, E_PAD, D)[:, :, :B2, :]
